# Initial kernel scaffold; baseline (speedup 1.0000x reference)
import jax
import jax.numpy as jnp
from jax.experimental import pallas as pl


def kernel(ego_embeddings, edge_index, edge_weight, W1, b1, g1, be1, W2, b2, g2, be2):
    raise NotImplementedError("write your pallas kernel here")


# speedup vs baseline: 3.3189x; 3.3189x over previous
import functools

import jax
import jax.numpy as jnp
from jax import lax
from jax.experimental import pallas as pl
from jax.experimental.pallas import tpu as pltpu
from jax.experimental.pallas import tpu_sc as plsc

N = 10000
D = 128
E = 320000
K = 128
LANES = 16
GROUPS = D // LANES


def _sc_aggregate_fn(nc, ns, chunks_per_worker):
    nw = nc * ns
    epw = chunks_per_worker * K
    rows_per_tile = -(-N // (ns * 8)) * 8
    n_pad = ns * rows_per_tile

    mesh = plsc.VectorSubcoreMesh(core_axis_name="c", subcore_axis_name="s",
                                  num_cores=nc, num_subcores=ns)

    @functools.partial(
        pl.kernel,
        out_type=jax.ShapeDtypeStruct((2 * n_pad, D), jnp.float32),
        mesh=mesh,
        scratch_types=[
            pltpu.VMEM((K,), jnp.int32),
            pltpu.VMEM((K,), jnp.int32),
            pltpu.VMEM((K,), jnp.float32),
            pltpu.VMEM((K, D), jnp.float32),
            pltpu.VMEM_SHARED((n_pad, D), jnp.float32),
            pltpu.SemaphoreType.DMA,
        ],
    )
    def agg(x_hbm, src_hbm, dst_hbm, w_hbm, out_hbm,
            src_v, dst_v, w_v, rows_v, acc, gsem):
        cid = lax.axis_index("c")
        sid = lax.axis_index("s")
        wid = sid * nc + cid

        @pl.loop(0, K)
        def _zero_buf(i):
            for j in range(GROUPS):
                rows_v[i, pl.ds(j * LANES, LANES)] = jnp.zeros((LANES,), jnp.float32)

        row0 = sid * rows_per_tile
        done = 0
        while done < rows_per_tile:
            n = min(K, rows_per_tile - done)
            pltpu.sync_copy(rows_v.at[pl.ds(0, n)], acc.at[pl.ds(row0 + done, n)])
            done += n
        plsc.subcore_barrier()

        base = wid * epw

        @pl.loop(0, chunks_per_worker)
        def _chunk(c):
            off = base + c * K
            pltpu.sync_copy(src_hbm.at[pl.ds(off, K)], src_v)
            pltpu.sync_copy(dst_hbm.at[pl.ds(off, K)], dst_v)
            pltpu.sync_copy(w_hbm.at[pl.ds(off, K)], w_v)
            pltpu.async_copy(x_hbm.at[src_v], rows_v, gsem).wait()

            @pl.loop(0, K // LANES)
            def _scale(ii):
                wv = w_v[pl.ds(ii * LANES, LANES)]
                for l in range(LANES):
                    w = wv[l]
                    i = ii * LANES + l
                    for j in range(GROUPS):
                        sl = pl.ds(j * LANES, LANES)
                        rows_v[i, sl] = rows_v[i, sl] * w

            pltpu.sync_copy(rows_v, acc.at[dst_v], add=True)

        plsc.subcore_barrier()

        pltpu.sync_copy(acc.at[pl.ds(row0, rows_per_tile)],
                        out_hbm.at[pl.ds(cid * n_pad + row0, rows_per_tile)])

    return agg, n_pad


def _dense_kernel(x_ref, p0_ref, p1_ref, w_ref, b_ref, g_ref, be_ref,
                  h_ref, y_ref):
    hi = x_ref[...] + p0_ref[...] + p1_ref[...]
    z = jnp.dot(hi, w_ref[...], preferred_element_type=jnp.float32) + b_ref[...]
    z = jnp.where(z >= 0, z, 0.01 * z)
    m = jnp.mean(z, axis=-1, keepdims=True)
    v = jnp.mean((z - m) ** 2, axis=-1, keepdims=True)
    h = (z - m) * lax.rsqrt(v + 1e-5) * g_ref[...] + be_ref[...]
    h_ref[...] = h
    nrm = jnp.sqrt(jnp.sum(h * h, axis=-1, keepdims=True))
    y_ref[...] = h / jnp.maximum(nrm, 1e-12)


def _dense_layer(x, p0, p1, W, b, g, be):
    blk = 2000
    grid = (N // blk,)
    row_spec = pl.BlockSpec((blk, D), lambda i: (i, 0))
    rep_spec = pl.BlockSpec((1, D), lambda i: (0, 0))
    return pl.pallas_call(
        _dense_kernel,
        grid=grid,
        in_specs=[row_spec, row_spec, row_spec,
                  pl.BlockSpec((D, D), lambda i: (0, 0)),
                  rep_spec, rep_spec, rep_spec],
        out_specs=[row_spec, row_spec],
        out_shape=[jax.ShapeDtypeStruct((N, D), jnp.float32),
                   jax.ShapeDtypeStruct((N, D), jnp.float32)],
    )(x, p0, p1, W, b.reshape(1, D), g.reshape(1, D), be.reshape(1, D))


def kernel(ego_embeddings, edge_index, edge_weight, W1, b1, g1, be1,
           W2, b2, g2, be2):
    info = plsc.get_sparse_core_info()
    nc, ns = info.num_cores, info.num_subcores
    nw = nc * ns
    chunks_per_worker = -(-E // (nw * K))
    e_pad = nw * chunks_per_worker * K

    src = edge_index[0]
    dst = edge_index[1]
    pad = e_pad - E
    if pad:
        src = jnp.concatenate([src, jnp.zeros((pad,), jnp.int32)])
        dst = jnp.concatenate([dst, jnp.zeros((pad,), jnp.int32)])
        edge_weight = jnp.concatenate([edge_weight, jnp.zeros((pad,), jnp.float32)])

    agg, n_pad = _sc_aggregate_fn(nc, ns, chunks_per_worker)

    def layer(x, W, b, g, be):
        part = agg(x, src, dst, edge_weight)
        return _dense_layer(x, part[:N], part[n_pad:n_pad + N], W, b, g, be)

    h1, y1 = layer(ego_embeddings, W1, b1, g1, be1)
    _, y2 = layer(h1, W2, b2, g2, be2)
    return jnp.concatenate([ego_embeddings, y1, y2], axis=1)
